# baseline (speedup 1.0000x reference)
import functools
import jax
import jax.numpy as jnp
from jax import lax
from jax.experimental import pallas as pl
from jax.experimental.pallas import tpu as pltpu
from jax.experimental.pallas import tpu_sc as plsc

N_ROWS = 320000
DIM = 128
HID = 128
NSEG = 1024
BLK = 2560
NSC = 16
NW = 32
SCHUNK = N_ROWS // NSC
NGRP = SCHUNK // 16
TROWS = 128
NTILES = N_ROWS // TROWS
LANE = 16


def _scores_body(x_ref, w1_ref, b1_ref, w2_ref, b2_ref, out_ref):
    x = x_ref[...]
    h = jnp.maximum(
        jnp.dot(x, w1_ref[...].T, preferred_element_type=jnp.float32,
                precision=lax.Precision.DEFAULT)
        + b1_ref[...],
        0.0,
    )
    g = lax.dot_general(
        w2_ref[...], h, (((1,), (1,)), ((), ())),
        preferred_element_type=jnp.float32,
        precision=lax.Precision.DEFAULT)
    out_ref[...] = g.reshape(1, 1, BLK) + b2_ref[0, 0]


def _gate_scores(x, W1, b1, W2, b2):
    nblk = N_ROWS // BLK
    out = pl.pallas_call(
        _scores_body,
        grid=(nblk,),
        in_specs=[
            pl.BlockSpec((BLK, DIM), lambda i: (i, 0)),
            pl.BlockSpec((HID, DIM), lambda i: (0, 0)),
            pl.BlockSpec((1, HID), lambda i: (0, 0)),
            pl.BlockSpec((1, HID), lambda i: (0, 0)),
            pl.BlockSpec((1, 1), lambda i: (0, 0), memory_space=pltpu.SMEM),
        ],
        out_specs=pl.BlockSpec((1, 1, BLK), lambda i: (i, 0, 0)),
        out_shape=jax.ShapeDtypeStruct((nblk, 1, BLK), jnp.float32),
    )(x, W1, b1.reshape(1, HID), W2, b2.reshape(1, 1))
    return out.reshape(N_ROWS)


def _pool_body(x_hbm, batch_hbm, scores_hbm, outpart_hbm,
               sc_buf, bt_buf, mtab, ztab, mrow, zrow,
               mfin, zinv, pbufm, pbufz,
               xbuf, idxbuf, sbuf, wbuf, insem, scsem,
               acc, mshr, zshr):
    cid = lax.axis_index("c")
    sid = lax.axis_index("s")
    wid = sid * 2 + cid

    lanes = jnp.arange(LANE, dtype=jnp.int32)

    def zero_body(i, _):
        z16 = jnp.zeros((LANE,), jnp.float32)
        mtab[pl.ds(i * LANE, LANE)] = z16
        ztab[pl.ds(i * LANE, LANE)] = z16
        return 0
    lax.fori_loop(0, NSEG * LANE // LANE, zero_body, 0)

    base = sid * SCHUNK
    pltpu.sync_copy(scores_hbm.at[pl.ds(base, SCHUNK)], sc_buf)
    pltpu.sync_copy(batch_hbm.at[pl.ds(base, SCHUNK)], bt_buf)

    def grp_body(g, _):
        idx = bt_buf[pl.ds(g * LANE, LANE)]
        s = sc_buf[pl.ds(g * LANE, LANE)]
        flat = idx * LANE + lanes
        mo = plsc.load_gather(mtab, [flat])
        zo = plsc.load_gather(ztab, [flat])
        mn = jnp.maximum(mo, s)
        zn = zo * jnp.exp(mo - mn) + jnp.exp(s - mn)
        plsc.store_scatter(mtab, [flat], mn)
        plsc.store_scatter(ztab, [flat], zn)
        return 0
    lax.fori_loop(0, NGRP, grp_body, 0)

    def red_body(sg, _):
        idx0 = (lanes + sg * LANE) * LANE
        m_acc = plsc.load_gather(mtab, [idx0])
        for c in range(1, LANE):
            m_acc = jnp.maximum(m_acc, plsc.load_gather(mtab, [idx0 + c]))
        z_acc = jnp.zeros((LANE,), jnp.float32)
        for c in range(LANE):
            mv = plsc.load_gather(mtab, [idx0 + c])
            zv = plsc.load_gather(ztab, [idx0 + c])
            z_acc = z_acc + zv * jnp.exp(mv - m_acc)
        mrow[pl.ds(sg * LANE, LANE)] = m_acc
        zrow[pl.ds(sg * LANE, LANE)] = z_acc
        return 0
    lax.fori_loop(0, NSEG // LANE, red_body, 0)

    pltpu.sync_copy(mrow, mshr.at[sid])
    pltpu.sync_copy(zrow, zshr.at[sid])

    def zx_body(i, _):
        for c in range(DIM // LANE):
            xbuf[0, i, pl.ds(c * LANE, LANE)] = jnp.zeros((LANE,),
                                                          jnp.float32)
        return 0
    lax.fori_loop(0, NSEG // NSC, zx_body, 0)
    pltpu.sync_copy(xbuf.at[0, pl.ds(0, NSEG // NSC)],
                    acc.at[pl.ds(sid * (NSEG // NSC), NSEG // NSC)])
    plsc.subcore_barrier()

    CCH = 128
    for csi in range(NSEG // CCH):
        pltpu.sync_copy(mshr.at[:, pl.ds(csi * CCH, CCH)], pbufm)
        pltpu.sync_copy(zshr.at[:, pl.ds(csi * CCH, CCH)], pbufz)

        def comb_body(sg, _):
            m_acc = jnp.zeros((LANE,), jnp.float32)

            def max_body(w, m_acc):
                return jnp.maximum(m_acc, pbufm[w, pl.ds(sg * LANE, LANE)])
            m_acc = lax.fori_loop(0, NSC, max_body, m_acc)

            def sum_body(w, z_acc):
                mv = pbufm[w, pl.ds(sg * LANE, LANE)]
                zv = pbufz[w, pl.ds(sg * LANE, LANE)]
                return z_acc + zv * jnp.exp(mv - m_acc)
            z_acc = lax.fori_loop(0, NSC, sum_body,
                                  jnp.zeros((LANE,), jnp.float32))

            off = csi * CCH + sg * LANE
            mfin[pl.ds(off, LANE)] = m_acc
            zinv[pl.ds(off, LANE)] = 1.0 / (z_acc + 1e-9)
            return 0
        lax.fori_loop(0, CCH // LANE, comb_body, 0)

    nfull = NTILES // NW
    nrem = NTILES - nfull * NW
    ntiles_w = nfull + jnp.where(wid < nrem, 1, 0)

    def issue_in(t, b):
        row0 = t * TROWS
        pltpu.async_copy(x_hbm.at[pl.ds(row0, TROWS)], xbuf.at[b], insem)
        pltpu.async_copy(batch_hbm.at[pl.ds(row0, TROWS)], idxbuf.at[b],
                         insem)
        pltpu.async_copy(scores_hbm.at[pl.ds(row0, TROWS)], sbuf.at[b],
                         insem)

    def wait_in(t, b):
        row0 = t * TROWS
        pltpu.make_async_copy(x_hbm.at[pl.ds(row0, TROWS)], xbuf.at[b],
                              insem).wait()
        pltpu.make_async_copy(batch_hbm.at[pl.ds(row0, TROWS)],
                              idxbuf.at[b], insem).wait()
        pltpu.make_async_copy(scores_hbm.at[pl.ds(row0, TROWS)],
                              sbuf.at[b], insem).wait()

    def wait_scatter():
        pltpu.make_async_copy(xbuf.at[0], acc.at[idxbuf.at[0]],
                              scsem).wait()

    issue_in(wid, 0)

    def tile_body(k, _):
        b = lax.rem(k, 2)
        t = wid + k * NW

        wait_in(t, b)

        for g in range(TROWS // LANE):
            idx = idxbuf[b, pl.ds(g * LANE, LANE)]
            s = sbuf[b, pl.ds(g * LANE, LANE)]
            m = plsc.load_gather(mfin, [idx])
            zi = plsc.load_gather(zinv, [idx])
            wbuf[pl.ds(g * LANE, LANE)] = jnp.exp(s - m) * zi

        def row_body(jj, _):
            j0 = jj * 4
            for u in range(4):
                wj = plsc.load_gather(
                    wbuf, [jnp.full((LANE,), 0, jnp.int32) + (j0 + u)])
                for c in range(DIM // LANE):
                    xbuf[b, j0 + u, pl.ds(c * LANE, LANE)] = (
                        xbuf[b, j0 + u, pl.ds(c * LANE, LANE)] * wj)
            return 0
        lax.fori_loop(0, TROWS // 4, row_body, 0)

        @pl.when(k > 0)
        def _():
            wait_scatter()

        @pl.when(k + 1 < ntiles_w)
        def _():
            issue_in(wid + (k + 1) * NW, 1 - b)

        pltpu.async_copy(xbuf.at[b], acc.at[idxbuf.at[b]], scsem,
                         add=True)
        return 0
    lax.fori_loop(0, ntiles_w, tile_body, 0)
    wait_scatter()

    plsc.subcore_barrier()
    pltpu.sync_copy(acc.at[pl.ds(sid * (NSEG // NSC), NSEG // NSC)],
                    outpart_hbm.at[cid, pl.ds(sid * (NSEG // NSC),
                                              NSEG // NSC)])


def _pool(x, batch, scores):
    mesh = plsc.VectorSubcoreMesh(core_axis_name="c", subcore_axis_name="s")
    f = functools.partial(
        pl.kernel,
        mesh=mesh,
        compiler_params=pltpu.CompilerParams(needs_layout_passes=False),
        out_type=jax.ShapeDtypeStruct((2, NSEG, DIM), jnp.float32),
        scratch_types=[
            pltpu.VMEM((SCHUNK,), jnp.float32),
            pltpu.VMEM((SCHUNK,), jnp.int32),
            pltpu.VMEM((NSEG * LANE,), jnp.float32),
            pltpu.VMEM((NSEG * LANE,), jnp.float32),
            pltpu.VMEM((NSEG,), jnp.float32),
            pltpu.VMEM((NSEG,), jnp.float32),
            pltpu.VMEM((NSEG,), jnp.float32),
            pltpu.VMEM((NSEG,), jnp.float32),
            pltpu.VMEM((NSC, 128), jnp.float32),
            pltpu.VMEM((NSC, 128), jnp.float32),
            pltpu.VMEM((2, TROWS, DIM), jnp.float32),
            pltpu.VMEM((2, TROWS), jnp.int32),
            pltpu.VMEM((2, TROWS), jnp.float32),
            pltpu.VMEM((TROWS,), jnp.float32),
            pltpu.SemaphoreType.DMA,
            pltpu.SemaphoreType.DMA,
            pltpu.VMEM_SHARED((NSEG, DIM), jnp.float32),
            pltpu.VMEM_SHARED((NSC, NSEG), jnp.float32),
            pltpu.VMEM_SHARED((NSC, NSEG), jnp.float32),
        ],
    )(_pool_body)
    return f(x, batch, scores)


def _sum_body(p_ref, out_ref):
    out_ref[...] = p_ref[0] + p_ref[1]


def _sum_partials(outpart):
    return pl.pallas_call(
        _sum_body,
        out_shape=jax.ShapeDtypeStruct((NSEG, DIM), jnp.float32),
    )(outpart)


def kernel(x, batch, W1, b1, W2, b2):
    scores = _gate_scores(x, W1, b1, W2, b2)
    outpart = _pool(x, batch, scores)
    return _sum_partials(outpart)


# speedup vs baseline: 1.2101x; 1.2101x over previous
import functools
import jax
import jax.numpy as jnp
from jax import lax
from jax.experimental import pallas as pl
from jax.experimental.pallas import tpu as pltpu
from jax.experimental.pallas import tpu_sc as plsc

N_ROWS = 320000
DIM = 128
HID = 128
NSEG = 1024
BLK = 5000
NSC = 16
NW = 32
SCHUNK = N_ROWS // NSC
NGRP = SCHUNK // 16
TROWS = 128
NTILES = N_ROWS // TROWS
LANE = 16


def _scores_body(x_ref, w1_ref, b1_ref, w2_ref, b2_ref, out_ref):
    x = x_ref[...]
    h = jnp.maximum(
        jnp.dot(x, w1_ref[...].T, preferred_element_type=jnp.float32,
                precision=lax.Precision.DEFAULT)
        + b1_ref[...],
        0.0,
    )
    g = lax.dot_general(
        w2_ref[...], h, (((1,), (1,)), ((), ())),
        preferred_element_type=jnp.float32,
        precision=lax.Precision.DEFAULT)
    out_ref[...] = g.reshape(1, 1, BLK) + b2_ref[0, 0]


def _gate_scores(x, W1, b1, W2, b2):
    nblk = N_ROWS // BLK
    out = pl.pallas_call(
        _scores_body,
        grid=(nblk,),
        in_specs=[
            pl.BlockSpec((BLK, DIM), lambda i: (i, 0)),
            pl.BlockSpec((HID, DIM), lambda i: (0, 0)),
            pl.BlockSpec((1, HID), lambda i: (0, 0)),
            pl.BlockSpec((1, HID), lambda i: (0, 0)),
            pl.BlockSpec((1, 1), lambda i: (0, 0), memory_space=pltpu.SMEM),
        ],
        out_specs=pl.BlockSpec((1, 1, BLK), lambda i: (i, 0, 0)),
        out_shape=jax.ShapeDtypeStruct((nblk, 1, BLK), jnp.float32),
    )(x, W1, b1.reshape(1, HID), W2, b2.reshape(1, 1))
    return out.reshape(N_ROWS)


def _pool_body(x_hbm, batch_hbm, scores_hbm, outpart_hbm,
               sc_buf, bt_buf, mtab, ztab, mrow, zrow,
               mfin, zinv, pbufm, pbufz,
               xbuf, idxbuf, sbuf, wbuf, insem,
               acc, mshr, zshr):
    cid = lax.axis_index("c")
    sid = lax.axis_index("s")
    wid = sid * 2 + cid

    lanes = jnp.arange(LANE, dtype=jnp.int32)

    def zero_body(i, _):
        z16 = jnp.zeros((LANE,), jnp.float32)
        mtab[pl.ds(i * LANE, LANE)] = z16
        ztab[pl.ds(i * LANE, LANE)] = z16
        return 0
    lax.fori_loop(0, NSEG * LANE // LANE, zero_body, 0)

    base = sid * SCHUNK
    pltpu.sync_copy(scores_hbm.at[pl.ds(base, SCHUNK)], sc_buf)
    pltpu.sync_copy(batch_hbm.at[pl.ds(base, SCHUNK)], bt_buf)

    def grp_body(g, _):
        idx = bt_buf[pl.ds(g * LANE, LANE)]
        s = sc_buf[pl.ds(g * LANE, LANE)]
        flat = idx * LANE + lanes
        mo = plsc.load_gather(mtab, [flat])
        zo = plsc.load_gather(ztab, [flat])
        mn = jnp.maximum(mo, s)
        zn = zo * jnp.exp(mo - mn) + jnp.exp(s - mn)
        plsc.store_scatter(mtab, [flat], mn)
        plsc.store_scatter(ztab, [flat], zn)
        return 0
    lax.fori_loop(0, NGRP, grp_body, 0)

    def red_body(sg, _):
        idx0 = (lanes + sg * LANE) * LANE
        m_acc = plsc.load_gather(mtab, [idx0])
        for c in range(1, LANE):
            m_acc = jnp.maximum(m_acc, plsc.load_gather(mtab, [idx0 + c]))
        z_acc = jnp.zeros((LANE,), jnp.float32)
        for c in range(LANE):
            mv = plsc.load_gather(mtab, [idx0 + c])
            zv = plsc.load_gather(ztab, [idx0 + c])
            z_acc = z_acc + zv * jnp.exp(mv - m_acc)
        mrow[pl.ds(sg * LANE, LANE)] = m_acc
        zrow[pl.ds(sg * LANE, LANE)] = z_acc
        return 0
    lax.fori_loop(0, NSEG // LANE, red_body, 0)

    pltpu.sync_copy(mrow, mshr.at[sid])
    pltpu.sync_copy(zrow, zshr.at[sid])

    def zx_body(i, _):
        for c in range(DIM // LANE):
            xbuf[0, i, pl.ds(c * LANE, LANE)] = jnp.zeros((LANE,),
                                                          jnp.float32)
        return 0
    lax.fori_loop(0, NSEG // NSC, zx_body, 0)
    pltpu.sync_copy(xbuf.at[0, pl.ds(0, NSEG // NSC)],
                    acc.at[pl.ds(sid * (NSEG // NSC), NSEG // NSC)])
    plsc.subcore_barrier()

    CCH = 128
    for csi in range(NSEG // CCH):
        pltpu.sync_copy(mshr.at[:, pl.ds(csi * CCH, CCH)], pbufm)
        pltpu.sync_copy(zshr.at[:, pl.ds(csi * CCH, CCH)], pbufz)

        def comb_body(sg, _):
            m_acc = jnp.zeros((LANE,), jnp.float32)

            def max_body(w, m_acc):
                return jnp.maximum(m_acc, pbufm[w, pl.ds(sg * LANE, LANE)])
            m_acc = lax.fori_loop(0, NSC, max_body, m_acc)

            def sum_body(w, z_acc):
                mv = pbufm[w, pl.ds(sg * LANE, LANE)]
                zv = pbufz[w, pl.ds(sg * LANE, LANE)]
                return z_acc + zv * jnp.exp(mv - m_acc)
            z_acc = lax.fori_loop(0, NSC, sum_body,
                                  jnp.zeros((LANE,), jnp.float32))

            off = csi * CCH + sg * LANE
            mfin[pl.ds(off, LANE)] = m_acc
            zinv[pl.ds(off, LANE)] = 1.0 / (z_acc + 1e-9)
            return 0
        lax.fori_loop(0, CCH // LANE, comb_body, 0)

    nfull = NTILES // NW
    nrem = NTILES - nfull * NW
    ntiles_w = nfull + jnp.where(wid < nrem, 1, 0)

    def issue_in(t, b):
        row0 = t * TROWS
        pltpu.async_copy(x_hbm.at[pl.ds(row0, TROWS)], xbuf.at[b], insem)
        pltpu.async_copy(batch_hbm.at[pl.ds(row0, TROWS)], idxbuf.at[b],
                         insem)
        pltpu.async_copy(scores_hbm.at[pl.ds(row0, TROWS)], sbuf.at[b],
                         insem)

    def wait_in(t, b):
        row0 = t * TROWS
        pltpu.make_async_copy(x_hbm.at[pl.ds(row0, TROWS)], xbuf.at[b],
                              insem).wait()
        pltpu.make_async_copy(batch_hbm.at[pl.ds(row0, TROWS)],
                              idxbuf.at[b], insem).wait()
        pltpu.make_async_copy(scores_hbm.at[pl.ds(row0, TROWS)],
                              sbuf.at[b], insem).wait()

    issue_in(wid, 0)

    def tile_body(k, _):
        b = lax.rem(k, 2)
        t = wid + k * NW

        @pl.when(k + 1 < ntiles_w)
        def _():
            issue_in(wid + (k + 1) * NW, 1 - b)

        wait_in(t, b)

        for g in range(TROWS // LANE):
            idx = idxbuf[b, pl.ds(g * LANE, LANE)]
            s = sbuf[b, pl.ds(g * LANE, LANE)]
            m = plsc.load_gather(mfin, [idx])
            zi = plsc.load_gather(zinv, [idx])
            wbuf[pl.ds(g * LANE, LANE)] = jnp.exp(s - m) * zi

        def row_body(jj, _):
            j0 = jj * 4
            for u in range(4):
                wj = plsc.load_gather(
                    wbuf, [jnp.full((LANE,), 0, jnp.int32) + (j0 + u)])
                for c in range(DIM // LANE):
                    xbuf[b, j0 + u, pl.ds(c * LANE, LANE)] = (
                        xbuf[b, j0 + u, pl.ds(c * LANE, LANE)] * wj)
            return 0
        lax.fori_loop(0, TROWS // 4, row_body, 0)

        pltpu.sync_copy(xbuf.at[b], acc.at[idxbuf.at[b]], add=True)
        return 0
    lax.fori_loop(0, ntiles_w, tile_body, 0)

    plsc.subcore_barrier()
    pltpu.sync_copy(acc.at[pl.ds(sid * (NSEG // NSC), NSEG // NSC)],
                    outpart_hbm.at[cid, pl.ds(sid * (NSEG // NSC),
                                              NSEG // NSC)])


def _pool(x, batch, scores):
    mesh = plsc.VectorSubcoreMesh(core_axis_name="c", subcore_axis_name="s")
    f = functools.partial(
        pl.kernel,
        mesh=mesh,
        compiler_params=pltpu.CompilerParams(needs_layout_passes=False),
        out_type=jax.ShapeDtypeStruct((2, NSEG, DIM), jnp.float32),
        scratch_types=[
            pltpu.VMEM((SCHUNK,), jnp.float32),
            pltpu.VMEM((SCHUNK,), jnp.int32),
            pltpu.VMEM((NSEG * LANE,), jnp.float32),
            pltpu.VMEM((NSEG * LANE,), jnp.float32),
            pltpu.VMEM((NSEG,), jnp.float32),
            pltpu.VMEM((NSEG,), jnp.float32),
            pltpu.VMEM((NSEG,), jnp.float32),
            pltpu.VMEM((NSEG,), jnp.float32),
            pltpu.VMEM((NSC, 128), jnp.float32),
            pltpu.VMEM((NSC, 128), jnp.float32),
            pltpu.VMEM((2, TROWS, DIM), jnp.float32),
            pltpu.VMEM((2, TROWS), jnp.int32),
            pltpu.VMEM((2, TROWS), jnp.float32),
            pltpu.VMEM((TROWS,), jnp.float32),
            pltpu.SemaphoreType.DMA,
            pltpu.VMEM_SHARED((NSEG, DIM), jnp.float32),
            pltpu.VMEM_SHARED((NSC, NSEG), jnp.float32),
            pltpu.VMEM_SHARED((NSC, NSEG), jnp.float32),
        ],
    )(_pool_body)
    return f(x, batch, scores)


def _sum_body(p_ref, out_ref):
    out_ref[...] = p_ref[0] + p_ref[1]


def _sum_partials(outpart):
    return pl.pallas_call(
        _sum_body,
        out_shape=jax.ShapeDtypeStruct((NSEG, DIM), jnp.float32),
    )(outpart)


def kernel(x, batch, W1, b1, W2, b2):
    scores = _gate_scores(x, W1, b1, W2, b2)
    outpart = _pool(x, batch, scores)
    return _sum_partials(outpart)


# speedup vs baseline: 1.2127x; 1.0022x over previous
import functools
import jax
import jax.numpy as jnp
from jax import lax
from jax.experimental import pallas as pl
from jax.experimental.pallas import tpu as pltpu
from jax.experimental.pallas import tpu_sc as plsc

N_ROWS = 320000
DIM = 128
HID = 128
NSEG = 1024
BLK = 5000
NSC = 16
NW = 32
SCHUNK = N_ROWS // NSC
NGRP = SCHUNK // 16
TROWS = 128
NTILES = N_ROWS // TROWS
LANE = 16


def _scores_body(x_ref, w1_ref, b1_ref, w2_ref, b2_ref, out_ref):
    x = x_ref[...]
    h = jnp.maximum(
        jnp.dot(x, w1_ref[...].T, preferred_element_type=jnp.float32,
                precision=lax.Precision.DEFAULT)
        + b1_ref[...],
        0.0,
    )
    g = lax.dot_general(
        w2_ref[...], h, (((1,), (1,)), ((), ())),
        preferred_element_type=jnp.float32,
        precision=lax.Precision.DEFAULT)
    out_ref[...] = g.reshape(1, 1, BLK) + b2_ref[0, 0]


def _gate_scores(x, W1, b1, W2, b2):
    nblk = N_ROWS // BLK
    out = pl.pallas_call(
        _scores_body,
        grid=(nblk,),
        in_specs=[
            pl.BlockSpec((BLK, DIM), lambda i: (i, 0)),
            pl.BlockSpec((HID, DIM), lambda i: (0, 0)),
            pl.BlockSpec((1, HID), lambda i: (0, 0)),
            pl.BlockSpec((1, HID), lambda i: (0, 0)),
            pl.BlockSpec((1, 1), lambda i: (0, 0), memory_space=pltpu.SMEM),
        ],
        out_specs=pl.BlockSpec((1, 1, BLK), lambda i: (i, 0, 0)),
        out_shape=jax.ShapeDtypeStruct((nblk, 1, BLK), jnp.float32),
    )(x, W1, b1.reshape(1, HID), W2, b2.reshape(1, 1))
    return out.reshape(N_ROWS)


def _pool_body(x_hbm, batch_hbm, scores_hbm, outpart_hbm,
               sc_buf, bt_buf, mtab, ztab, mrow, zrow,
               mfin, zinv, pbufm, pbufz,
               xbuf, idxbuf, sbuf, wbuf, insem,
               acc, mshr, zshr):
    cid = lax.axis_index("c")
    sid = lax.axis_index("s")
    wid = sid * 2 + cid

    lanes = jnp.arange(LANE, dtype=jnp.int32)

    def zero_body(i, _):
        z16 = jnp.zeros((LANE,), jnp.float32)
        mtab[pl.ds(i * LANE, LANE)] = z16
        ztab[pl.ds(i * LANE, LANE)] = z16
        return 0
    lax.fori_loop(0, NSEG * LANE // LANE, zero_body, 0)

    base = sid * SCHUNK
    pltpu.sync_copy(scores_hbm.at[pl.ds(base, SCHUNK)], sc_buf)
    pltpu.sync_copy(batch_hbm.at[pl.ds(base, SCHUNK)], bt_buf)

    def grp_body(g, _):
        idx = bt_buf[pl.ds(g * LANE, LANE)]
        s = sc_buf[pl.ds(g * LANE, LANE)]
        flat = idx * LANE + lanes
        mo = plsc.load_gather(mtab, [flat])
        zo = plsc.load_gather(ztab, [flat])
        mn = jnp.maximum(mo, s)
        zn = zo * jnp.exp(mo - mn) + jnp.exp(s - mn)
        plsc.store_scatter(mtab, [flat], mn)
        plsc.store_scatter(ztab, [flat], zn)
        return 0
    lax.fori_loop(0, NGRP, grp_body, 0)

    def red_body(sg, _):
        idx0 = (lanes + sg * LANE) * LANE
        m_acc = plsc.load_gather(mtab, [idx0])
        for c in range(1, LANE):
            m_acc = jnp.maximum(m_acc, plsc.load_gather(mtab, [idx0 + c]))
        z_acc = jnp.zeros((LANE,), jnp.float32)
        for c in range(LANE):
            mv = plsc.load_gather(mtab, [idx0 + c])
            zv = plsc.load_gather(ztab, [idx0 + c])
            z_acc = z_acc + zv * jnp.exp(mv - m_acc)
        mrow[pl.ds(sg * LANE, LANE)] = m_acc
        zrow[pl.ds(sg * LANE, LANE)] = z_acc
        return 0
    lax.fori_loop(0, NSEG // LANE, red_body, 0)

    pltpu.sync_copy(mrow, mshr.at[sid])
    pltpu.sync_copy(zrow, zshr.at[sid])

    def zx_body(i, _):
        for c in range(DIM // LANE):
            xbuf[0, i, pl.ds(c * LANE, LANE)] = jnp.zeros((LANE,),
                                                          jnp.float32)
        return 0
    lax.fori_loop(0, NSEG // NSC, zx_body, 0)
    pltpu.sync_copy(xbuf.at[0, pl.ds(0, NSEG // NSC)],
                    acc.at[pl.ds(sid * (NSEG // NSC), NSEG // NSC)])
    plsc.subcore_barrier()

    CCH = 128
    for csi in range(NSEG // CCH):
        pltpu.sync_copy(mshr.at[:, pl.ds(csi * CCH, CCH)], pbufm)
        pltpu.sync_copy(zshr.at[:, pl.ds(csi * CCH, CCH)], pbufz)

        def comb_body(sg, _):
            m_acc = jnp.zeros((LANE,), jnp.float32)

            def max_body(w, m_acc):
                return jnp.maximum(m_acc, pbufm[w, pl.ds(sg * LANE, LANE)])
            m_acc = lax.fori_loop(0, NSC, max_body, m_acc)

            def sum_body(w, z_acc):
                mv = pbufm[w, pl.ds(sg * LANE, LANE)]
                zv = pbufz[w, pl.ds(sg * LANE, LANE)]
                return z_acc + zv * jnp.exp(mv - m_acc)
            z_acc = lax.fori_loop(0, NSC, sum_body,
                                  jnp.zeros((LANE,), jnp.float32))

            off = csi * CCH + sg * LANE
            mfin[pl.ds(off, LANE)] = m_acc
            zinv[pl.ds(off, LANE)] = 1.0 / (z_acc + 1e-9)
            return 0
        lax.fori_loop(0, CCH // LANE, comb_body, 0)

    nfull = NTILES // NW
    nrem = NTILES - nfull * NW
    ntiles_w = nfull + jnp.where(wid < nrem, 1, 0)

    def issue_in(t, b):
        row0 = t * TROWS
        pltpu.async_copy(x_hbm.at[pl.ds(row0, TROWS)], xbuf.at[b], insem)
        pltpu.async_copy(batch_hbm.at[pl.ds(row0, TROWS)], idxbuf.at[b],
                         insem)
        pltpu.async_copy(scores_hbm.at[pl.ds(row0, TROWS)], sbuf.at[b],
                         insem)

    def wait_in(t, b):
        row0 = t * TROWS
        pltpu.make_async_copy(x_hbm.at[pl.ds(row0, TROWS)], xbuf.at[b],
                              insem).wait()
        pltpu.make_async_copy(batch_hbm.at[pl.ds(row0, TROWS)],
                              idxbuf.at[b], insem).wait()
        pltpu.make_async_copy(scores_hbm.at[pl.ds(row0, TROWS)],
                              sbuf.at[b], insem).wait()

    issue_in(wid, 0)

    def tile_body(k, _):
        b = lax.rem(k, 2)
        t = wid + k * NW

        @pl.when(k + 1 < ntiles_w)
        def _():
            issue_in(wid + (k + 1) * NW, 1 - b)

        wait_in(t, b)

        for g in range(TROWS // LANE):
            idx = idxbuf[b, pl.ds(g * LANE, LANE)]
            s = sbuf[b, pl.ds(g * LANE, LANE)]
            m = plsc.load_gather(mfin, [idx])
            zi = plsc.load_gather(zinv, [idx])
            wbuf[pl.ds(g * LANE, LANE)] = jnp.exp(s - m) * zi

        def row_body(jj, _):
            j0 = jj * 8
            for u in range(8):
                wj = plsc.load_gather(
                    wbuf, [jnp.full((LANE,), 0, jnp.int32) + (j0 + u)])
                for c in range(DIM // LANE):
                    xbuf[b, j0 + u, pl.ds(c * LANE, LANE)] = (
                        xbuf[b, j0 + u, pl.ds(c * LANE, LANE)] * wj)
            return 0
        lax.fori_loop(0, TROWS // 8, row_body, 0)

        pltpu.sync_copy(xbuf.at[b], acc.at[idxbuf.at[b]], add=True)
        return 0
    lax.fori_loop(0, ntiles_w, tile_body, 0)

    plsc.subcore_barrier()
    pltpu.sync_copy(acc.at[pl.ds(sid * (NSEG // NSC), NSEG // NSC)],
                    outpart_hbm.at[cid, pl.ds(sid * (NSEG // NSC),
                                              NSEG // NSC)])


def _pool(x, batch, scores):
    mesh = plsc.VectorSubcoreMesh(core_axis_name="c", subcore_axis_name="s")
    f = functools.partial(
        pl.kernel,
        mesh=mesh,
        compiler_params=pltpu.CompilerParams(needs_layout_passes=False),
        out_type=jax.ShapeDtypeStruct((2, NSEG, DIM), jnp.float32),
        scratch_types=[
            pltpu.VMEM((SCHUNK,), jnp.float32),
            pltpu.VMEM((SCHUNK,), jnp.int32),
            pltpu.VMEM((NSEG * LANE,), jnp.float32),
            pltpu.VMEM((NSEG * LANE,), jnp.float32),
            pltpu.VMEM((NSEG,), jnp.float32),
            pltpu.VMEM((NSEG,), jnp.float32),
            pltpu.VMEM((NSEG,), jnp.float32),
            pltpu.VMEM((NSEG,), jnp.float32),
            pltpu.VMEM((NSC, 128), jnp.float32),
            pltpu.VMEM((NSC, 128), jnp.float32),
            pltpu.VMEM((2, TROWS, DIM), jnp.float32),
            pltpu.VMEM((2, TROWS), jnp.int32),
            pltpu.VMEM((2, TROWS), jnp.float32),
            pltpu.VMEM((TROWS,), jnp.float32),
            pltpu.SemaphoreType.DMA,
            pltpu.VMEM_SHARED((NSEG, DIM), jnp.float32),
            pltpu.VMEM_SHARED((NSC, NSEG), jnp.float32),
            pltpu.VMEM_SHARED((NSC, NSEG), jnp.float32),
        ],
    )(_pool_body)
    return f(x, batch, scores)


def _sum_body(p_ref, out_ref):
    out_ref[...] = p_ref[0] + p_ref[1]


def _sum_partials(outpart):
    return pl.pallas_call(
        _sum_body,
        out_shape=jax.ShapeDtypeStruct((NSEG, DIM), jnp.float32),
    )(outpart)


def kernel(x, batch, W1, b1, W2, b2):
    scores = _gate_scores(x, W1, b1, W2, b2)
    outpart = _pool(x, batch, scores)
    return _sum_partials(outpart)
